# baseline (speedup 1.0000x reference)
import functools

import jax
import jax.numpy as jnp
from jax import lax
from jax.experimental import pallas as pl
from jax.experimental.pallas import tpu as pltpu
from jax.experimental.pallas import tpu_sc as plsc


_TBLK = 1024
_K = 4


def _body(tok_hbm, tab_hbm, out_hbm, tab_v, in_v, out_v, tab_sem, in_sems, out_sems):
    n_chunks = tok_hbm.shape[0] // _TBLK
    tab_chunks = tab_hbm.shape[0] // _TBLK

    def in_copy(i, slot):
        return pltpu.make_async_copy(
            tok_hbm.at[pl.ds(i * _TBLK, _TBLK), :], in_v.at[slot], in_sems.at[slot])

    def out_copy(i, slot):
        return pltpu.make_async_copy(
            out_v.at[slot], out_hbm.at[pl.ds(i * _TBLK, _TBLK), :], out_sems.at[slot])

    pltpu.make_async_copy(tab_hbm, tab_v, tab_sem).start()
    for s in range(_K):
        in_copy(s, s).start()
    pltpu.make_async_copy(tab_hbm, tab_v, tab_sem).wait()

    def step(i, _):
        slot = jax.lax.rem(i, _K)
        in_copy(i, slot).wait()
        t = jax.lax.rem(i, tab_chunks) * _TBLK
        out_v[slot] = in_v[slot] + tab_v[pl.ds(t, _TBLK), :]
        out_copy(i, slot).start()

        @pl.when(i + _K < n_chunks)
        def _():
            in_copy(i + _K, slot).start()

        @pl.when(i >= _K - 1)
        def _():
            j = i - (_K - 1)
            out_copy(j, jax.lax.rem(j, _K)).wait()
        return 0

    jax.lax.fori_loop(0, n_chunks, step, 0)
    for r in range(_K - 1):
        idx = n_chunks - (_K - 1) + r
        out_copy(idx, idx % _K).wait()


def _tc_add(flat_tokens, pos_table):
    num_tokens, embed_dim = pos_table.shape
    return pl.pallas_call(
        _body,
        in_specs=[
            pl.BlockSpec(memory_space=pl.ANY),
            pl.BlockSpec(memory_space=pl.ANY),
        ],
        out_specs=pl.BlockSpec(memory_space=pl.ANY),
        out_shape=jax.ShapeDtypeStruct(flat_tokens.shape, flat_tokens.dtype),
        scratch_shapes=[
            pltpu.VMEM((num_tokens, embed_dim), jnp.float32),
            pltpu.VMEM((_K, _TBLK, embed_dim), jnp.float32),
            pltpu.VMEM((_K, _TBLK, embed_dim), jnp.float32),
            pltpu.SemaphoreType.DMA,
            pltpu.SemaphoreType.DMA((_K,)),
            pltpu.SemaphoreType.DMA((_K,)),
        ],
    )(flat_tokens, pos_table)


_SC_WORKERS = 32
_SC_ROWS = 8192 // _SC_WORKERS
_SC_CROWS = 64


def _sc_body(tok1d, tab1d, out1d, tok_v, tab_v):
    wid = lax.axis_index("s") * 2 + lax.axis_index("c")
    embed = 768
    chunk_elems = _SC_CROWS * embed
    base_row = wid * _SC_ROWS
    tok_off0 = 3 * 8192 * embed

    def do_chunk(c, _):
        row0 = base_row + c * _SC_CROWS
        pltpu.sync_copy(tok1d.at[pl.ds(tok_off0 + row0 * embed, chunk_elems)], tok_v)
        pltpu.sync_copy(tab1d.at[pl.ds(row0 * embed, chunk_elems)], tab_v)

        def add16(i, _):
            off = i * 64
            for u in range(4):
                s = off + u * 16
                tok_v[pl.ds(s, 16)] = tok_v[pl.ds(s, 16)] + tab_v[pl.ds(s, 16)]
            return 0

        lax.fori_loop(0, chunk_elems // 64, add16, 0)
        pltpu.sync_copy(tok_v, out1d.at[pl.ds(row0 * embed, chunk_elems)])
        return 0

    lax.fori_loop(0, _SC_ROWS // _SC_CROWS, do_chunk, 0)


def _sc_add_batch3(tok1d, tab1d):
    mesh = plsc.VectorSubcoreMesh(core_axis_name="c", subcore_axis_name="s")
    n = tab1d.shape[0]
    fn = functools.partial(
        pl.kernel,
        out_type=jax.ShapeDtypeStruct((n,), jnp.float32),
        mesh=mesh,
        scratch_types=[
            pltpu.VMEM((_SC_CROWS * 768,), jnp.float32),
            pltpu.VMEM((_SC_CROWS * 768,), jnp.float32),
        ],
    )(_sc_body)
    return fn(tok1d, tab1d)


def kernel(encoded_tokens, pos_table):
    batch, num_tokens, embed_dim = encoded_tokens.shape
    flat = encoded_tokens.reshape(batch * num_tokens, embed_dim)
    out = _tc_add(flat, pos_table).reshape(batch, num_tokens, embed_dim)
    sc_out = _sc_add_batch3(
        encoded_tokens.reshape(-1), pos_table.reshape(-1))
    return out.at[3, 0, 0].set(sc_out[0])


# speedup vs baseline: 3.3682x; 3.3682x over previous
import jax
import jax.numpy as jnp
from jax.experimental import pallas as pl
from jax.experimental.pallas import tpu as pltpu


_TBLK = 1024
_K = 4


def _body(tok_hbm, tab_hbm, out_hbm, tab_v, in_v, out_v, tab_sem, in_sems, out_sems):
    n_chunks = tok_hbm.shape[0] // _TBLK
    tab_chunks = tab_hbm.shape[0] // _TBLK

    def in_copy(i, slot):
        return pltpu.make_async_copy(
            tok_hbm.at[pl.ds(i * _TBLK, _TBLK), :], in_v.at[slot], in_sems.at[slot])

    def out_copy(i, slot):
        return pltpu.make_async_copy(
            out_v.at[slot], out_hbm.at[pl.ds(i * _TBLK, _TBLK), :], out_sems.at[slot])

    pltpu.make_async_copy(tab_hbm, tab_v, tab_sem).start()

    for s in range(_K):
        in_copy(s, s).start()

    pltpu.make_async_copy(tab_hbm, tab_v, tab_sem).wait()

    def step(i, _):
        slot = jax.lax.rem(i, _K)
        in_copy(i, slot).wait()
        t = jax.lax.rem(i, tab_chunks) * _TBLK
        out_v[slot] = in_v[slot] + tab_v[pl.ds(t, _TBLK), :]
        out_copy(i, slot).start()

        @pl.when(i + _K < n_chunks)
        def _():
            in_copy(i + _K, slot).start()

        @pl.when(i >= _K - 1)
        def _():
            j = i - (_K - 1)
            out_copy(j, jax.lax.rem(j, _K)).wait()
        return 0

    jax.lax.fori_loop(0, n_chunks, step, 0)

    for r in range(_K - 1):
        idx = n_chunks - (_K - 1) + r
        out_copy(idx, idx % _K).wait()


def kernel(encoded_tokens, pos_table):
    batch, num_tokens, embed_dim = encoded_tokens.shape
    flat = encoded_tokens.reshape(batch * num_tokens, embed_dim)
    out = pl.pallas_call(
        _body,
        in_specs=[
            pl.BlockSpec(memory_space=pl.ANY),
            pl.BlockSpec(memory_space=pl.ANY),
        ],
        out_specs=pl.BlockSpec(memory_space=pl.ANY),
        out_shape=jax.ShapeDtypeStruct(flat.shape, flat.dtype),
        scratch_shapes=[
            pltpu.VMEM((num_tokens, embed_dim), jnp.float32),
            pltpu.VMEM((_K, _TBLK, embed_dim), jnp.float32),
            pltpu.VMEM((_K, _TBLK, embed_dim), jnp.float32),
            pltpu.SemaphoreType.DMA,
            pltpu.SemaphoreType.DMA((_K,)),
            pltpu.SemaphoreType.DMA((_K,)),
        ],
    )(flat, pos_table)
    return out.reshape(batch, num_tokens, embed_dim)


# speedup vs baseline: 3.3754x; 1.0021x over previous
import jax
import jax.numpy as jnp
from jax.experimental import pallas as pl
from jax.experimental.pallas import tpu as pltpu


_TBLK = 1024
_K = 4
_H = _TBLK // 2


def _body(tok_hbm, tab_hbm, out_hbm, tab_v, in_v, out_v, tab_sem, in_sems, out_sems):
    n_chunks = tok_hbm.shape[0] // _TBLK
    tab_chunks = tab_hbm.shape[0] // _TBLK

    def in_copy(i, slot):
        return pltpu.make_async_copy(
            tok_hbm.at[pl.ds(i * _TBLK, _TBLK), :], in_v.at[slot], in_sems.at[slot])

    def out_copies(i, slot):
        return [
            pltpu.make_async_copy(
                out_v.at[slot, pl.ds(h * _H, _H), :],
                out_hbm.at[pl.ds(i * _TBLK + h * _H, _H), :],
                out_sems.at[slot, h])
            for h in range(2)
        ]

    def out_start(i, slot):
        for c in out_copies(i, slot):
            c.start()

    def out_wait(i, slot):
        for c in out_copies(i, slot):
            c.wait()

    pltpu.make_async_copy(tab_hbm, tab_v, tab_sem).start()

    for s in range(_K):
        in_copy(s, s).start()

    pltpu.make_async_copy(tab_hbm, tab_v, tab_sem).wait()

    def step(i, _):
        slot = jax.lax.rem(i, _K)
        in_copy(i, slot).wait()
        t = jax.lax.rem(i, tab_chunks) * _TBLK
        out_v[slot] = in_v[slot] + tab_v[pl.ds(t, _TBLK), :]
        out_start(i, slot)

        @pl.when(i + _K < n_chunks)
        def _():
            in_copy(i + _K, slot).start()

        @pl.when(i >= _K - 1)
        def _():
            j = i - (_K - 1)
            out_wait(j, jax.lax.rem(j, _K))
        return 0

    jax.lax.fori_loop(0, n_chunks, step, 0)

    for r in range(_K - 1):
        idx = n_chunks - (_K - 1) + r
        out_wait(idx, idx % _K)


def kernel(encoded_tokens, pos_table):
    batch, num_tokens, embed_dim = encoded_tokens.shape
    flat = encoded_tokens.reshape(batch * num_tokens, embed_dim)
    out = pl.pallas_call(
        _body,
        in_specs=[
            pl.BlockSpec(memory_space=pl.ANY),
            pl.BlockSpec(memory_space=pl.ANY),
        ],
        out_specs=pl.BlockSpec(memory_space=pl.ANY),
        out_shape=jax.ShapeDtypeStruct(flat.shape, flat.dtype),
        scratch_shapes=[
            pltpu.VMEM((num_tokens, embed_dim), jnp.float32),
            pltpu.VMEM((_K, _TBLK, embed_dim), jnp.float32),
            pltpu.VMEM((_K, _TBLK, embed_dim), jnp.float32),
            pltpu.SemaphoreType.DMA,
            pltpu.SemaphoreType.DMA((_K,)),
            pltpu.SemaphoreType.DMA((_K, 2)),
        ],
    )(flat, pos_table)
    return out.reshape(batch, num_tokens, embed_dim)
